# baseline (speedup 1.0000x reference)
import functools

import numpy as np
import jax
import jax.numpy as jnp
from jax import lax
from jax.experimental import pallas as pl
from jax.experimental.pallas import tpu as pltpu
from jax.experimental.pallas import tpu_sc as plsc

_RATIOS = (0.5, 1.0, 2.0)
_SCALES = (2, 4, 8)
_IMG = 1920.0
_L = 16
_NC = 2
_NS = 16
_N = 4
_M = 64
_K = 9
_G = 60
_CELLS = _G * _G
_CP = 4096
_A = _CELLS * _K
_TCELL = _CP // _NS
_CH = _TCELL * _K
_NCHUNK = _CH // _L
_TR = 72
_TCL = 450


def _gen_anchors(stride, G):
    ws, hs = [], []
    for r in _RATIOS:
        for s in _SCALES:
            size = stride * s
            ws.append(size * np.sqrt(1.0 / r))
            hs.append(size * np.sqrt(r))
    ws = np.asarray(ws, np.float32)
    hs = np.asarray(hs, np.float32)
    ctr = (np.arange(G, dtype=np.float32) + 0.5) * stride
    cy, cx = np.meshgrid(ctr, ctr, indexing='ij')
    cx = cx[:, :, None]
    cy = cy[:, :, None]
    x1 = cx - ws / 2.0
    y1 = cy - hs / 2.0
    x2 = cx + ws / 2.0
    y2 = cy + hs / 2.0
    return np.stack([x1, y1, x2, y2], axis=-1).astype(np.float32)


def _vlog(x):
    b = lax.bitcast_convert_type(x, jnp.int32)
    e = (b >> 23) - 127
    m = lax.bitcast_convert_type((b & 0x7FFFFF) | 0x3F800000, jnp.float32)
    big = m > 1.4142135623730951
    m = jnp.where(big, m * 0.5, m)
    ef = jnp.where(big, e + 1, e).astype(jnp.float32)
    s = (m - 1.0) / (m + 1.0)
    z = s * s
    p = 1.0 / 3 + z * (1.0 / 5 + z * (1.0 / 7 + z * (1.0 / 9)))
    return (2.0 * s) * (1.0 + z * p) + ef * 0.6931471805599453


def _tc_match_kernel(anc_ref, dlt_ref, gt_ref, mat_ref):
    rx1 = jnp.clip(anc_ref[0] + dlt_ref[0, 0], 0.0, _IMG)
    ry1 = jnp.clip(anc_ref[1] + dlt_ref[0, 1], 0.0, _IMG)
    rx2 = jnp.clip(anc_ref[2] + dlt_ref[0, 2], 0.0, _IMG)
    ry2 = jnp.clip(anc_ref[3] + dlt_ref[0, 3], 0.0, _IMG)
    area_r = (rx2 - rx1) * (ry2 - ry1)
    num_b = jnp.full((_TR, _TCL), -1.0, jnp.float32)
    den_b = jnp.full((_TR, _TCL), 1.0, jnp.float32)
    bidx = jnp.zeros((_TR, _TCL), jnp.int32)
    for m in range(_M):
        gx1 = gt_ref[0, m // 128, m % 128]
        gy1 = gt_ref[0, (_M + m) // 128, (_M + m) % 128]
        gx2 = gt_ref[0, (2 * _M + m) // 128, (2 * _M + m) % 128]
        gy2 = gt_ref[0, (3 * _M + m) // 128, (3 * _M + m) % 128]
        garea = (gx2 - gx1) * (gy2 - gy1)
        xx1 = jnp.maximum(rx1, gx1)
        yy1 = jnp.maximum(ry1, gy1)
        xx2 = jnp.minimum(rx2, gx2)
        yy2 = jnp.minimum(ry2, gy2)
        w = jnp.maximum(xx2 - xx1, 0.0)
        h = jnp.maximum(yy2 - yy1, 0.0)
        inter = w * h
        den = area_r + garea - inter + 1e-9
        upd = inter * den_b > num_b * den
        num_b = jnp.where(upd, inter, num_b)
        den_b = jnp.where(upd, den, den_b)
        bidx = jnp.where(upd, jnp.int32(m), bidx)
    best = num_b / den_b
    mat_ref[0] = jnp.where(best >= 0.4, bidx,
                           jnp.where(best < 0.1, jnp.int32(-1), jnp.int32(-2)))


_tc_match = pl.pallas_call(
    _tc_match_kernel,
    grid=(_N,),
    in_specs=[
        pl.BlockSpec((4, _TR, _TCL), lambda n: (0, 0, 0)),
        pl.BlockSpec((1, 4, _TR, _TCL), lambda n: (n, 0, 0, 0)),
        pl.BlockSpec((1, 8, 128), lambda n: (n, 0, 0)),
    ],
    out_specs=pl.BlockSpec((1, _TR, _TCL), lambda n: (n, 0, 0)),
    out_shape=jax.ShapeDtypeStruct((_N, _TR, _TCL), jnp.int32),
)


_mesh = plsc.VectorSubcoreMesh(core_axis_name="c", subcore_axis_name="s",
                               num_cores=_NC)


@functools.partial(
    pl.kernel,
    mesh=_mesh,
    compiler_params=pltpu.CompilerParams(needs_layout_passes=False),
    out_type=jax.ShapeDtypeStruct((_NC, _L), jnp.float32),
    scratch_types=[
        pltpu.VMEM((_K * _TCELL,), jnp.float32),
        pltpu.VMEM((4 * _K * _TCELL,), jnp.float32),
        pltpu.VMEM((4, _CH), jnp.float32),
        pltpu.VMEM((4 * _M,), jnp.float32),
        pltpu.VMEM((_K * _TCELL,), jnp.int32),
        pltpu.VMEM((_L,), jnp.float32),
        pltpu.VMEM((_NS * _L,), jnp.float32),
        pltpu.VMEM((_NS * _L,), jnp.float32),
        pltpu.VMEM_SHARED((_NS * _L,), jnp.float32),
        pltpu.VMEM_SHARED((_NS * _L,), jnp.float32),
    ],
)
def _sc_kernel(cls_h, dlt_h, anc_h, gtt_h, mat_h, out_h,
               cls_v, dlt_v, anc_v, gt_v, mat_v, row_v,
               cnt_v, prt_v, cnt_sh, prt_sh):
    c = lax.axis_index("c")
    s = lax.axis_index("s")
    base_a = s * _CH
    c0 = s * _TCELL
    lane = lax.iota(jnp.int32, _L)

    for j in range(4):
        pltpu.sync_copy(anc_h.at[pl.ds(j * (_CP * _K) + base_a, _CH)],
                        anc_v.at[j])

    def image_step(t_img, accs):
        acc_cls, acc_bb, acc_bg, acc_fg, acc_pr = accs
        img = c * 2 + t_img
        is_last = img == _N - 1

        for k in range(_K):
            pltpu.sync_copy(
                cls_h.at[pl.ds((img * _K + k) * _CP + c0, _TCELL)],
                cls_v.at[pl.ds(k * _TCELL, _TCELL)])
            pltpu.sync_copy(
                mat_h.at[pl.ds((img * _K + k) * _CP + c0, _TCELL)],
                mat_v.at[pl.ds(k * _TCELL, _TCELL)])
        for r in range(4 * _K):
            pltpu.sync_copy(
                dlt_h.at[pl.ds((img * 4 * _K + r) * _CP + c0, _TCELL)],
                dlt_v.at[pl.ds(r * _TCELL, _TCELL)])
        for j in range(4):
            pltpu.sync_copy(gtt_h.at[pl.ds((img * 4 + j) * _M, _M)],
                            gt_v.at[pl.ds(j * _M, _M)])

        def perm_idx(o):
            ti = o + lane
            tf = ti.astype(jnp.float32)
            cellr = ((tf + 0.5) * (1.0 / 9.0)).astype(jnp.int32)
            ki = ti - cellr * 9
            ci = ki * _TCELL + cellr
            di = ki * (4 * _TCELL) + cellr
            return ci, di

        def chunk_a(t, carry):
            cp, cn = carry
            o = t * _L
            ci, _ = perm_idx(o)
            match = plsc.load_gather(mat_v, [ci])
            valid = (base_a + o + lane) < _A
            cp = cp + jnp.sum(jnp.where((match >= 0) & valid, 1.0, 0.0))
            cn = cn + jnp.sum(jnp.where((match == -1) & valid, 1.0, 0.0))
            return cp, cn

        cp, cn = lax.fori_loop(0, _NCHUNK, chunk_a,
                               (jnp.float32(0.0), jnp.float32(0.0)))

        row_v[...] = jnp.where(lane == 0, cp, jnp.where(lane == 1, cn, 0.0))
        pltpu.sync_copy(row_v, cnt_sh.at[pl.ds(s * _L, _L)])
        plsc.subcore_barrier()
        pltpu.sync_copy(cnt_sh, cnt_v)

        lane16 = lane * _L
        cpos_all = plsc.load_gather(cnt_v, [lane16])
        cneg_all = plsc.load_gather(cnt_v, [lane16 + 1])
        before = lane < s
        base_pos = jnp.sum(jnp.where(before, cpos_all, 0.0))
        base_neg = jnp.sum(jnp.where(before, cneg_all, 0.0))
        totpos = jnp.sum(cpos_all)
        totneg = jnp.sum(cneg_all)
        nselp = jnp.minimum(totpos, 128.0)
        nseln = jnp.minimum(totneg, 60.0)
        denom = jnp.maximum(nselp + nseln, 1.0)
        capp = 128.0 - nselp
        capn = 60.0 - nseln

        def chunk_b(t, carry):
            run_p, run_n, a_bce, a_bb, a_pr = carry
            o = t * _L
            ci, di = perm_idx(o)
            match = plsc.load_gather(mat_v, [ci])
            valid = (base_a + o + lane) < _A
            pos = (match >= 0) & valid
            neg = (match == -1) & valid
            posf = jnp.where(pos, 1.0, 0.0)
            negf = jnp.where(neg, 1.0, 0.0)
            rp = run_p + (plsc.cumsum(posf) - posf)
            rn = run_n + (plsc.cumsum(negf) - negf)
            g = (base_a + o + lane).astype(jnp.float32)
            gr_pos = base_pos + rp
            gr_neg = base_neg + rn
            gr_npos = g - gr_pos
            gr_nneg = g - gr_neg
            selp = pos & (gr_pos < 128.0)
            seln = neg & (gr_neg < 60.0)
            padp = jnp.logical_and((~pos) & (gr_npos < capp), is_last)
            padn = jnp.logical_and((~neg) & (gr_nneg < capn), is_last)
            selp_l = jnp.logical_and(selp, is_last)
            seln_l = jnp.logical_and(seln, is_last)

            p = plsc.load_gather(cls_v, [ci])
            u = _vlog(1.0 + jnp.exp(-jnp.abs(p)))
            a_bce = a_bce + jnp.where(selp, jnp.maximum(-p, 0.0) + u, 0.0) \
                          + jnp.where(seln, jnp.maximum(p, 0.0) + u, 0.0)
            a_pr = a_pr + jnp.where(selp_l, p, 0.0) + jnp.where(padp, p, 0.0) \
                        + jnp.where(seln_l, p, 0.0) + jnp.where(padn, p, 0.0)

            midx = jnp.maximum(match, 0)
            gx1 = plsc.load_gather(gt_v, [midx])
            gy1 = plsc.load_gather(gt_v, [midx + _M])
            gx2 = plsc.load_gather(gt_v, [midx + 2 * _M])
            gy2 = plsc.load_gather(gt_v, [midx + 3 * _M])
            ax1 = anc_v[0, pl.ds(o, _L)]
            ay1 = anc_v[1, pl.ds(o, _L)]
            ax2 = anc_v[2, pl.ds(o, _L)]
            ay2 = anc_v[3, pl.ds(o, _L)]
            d0 = plsc.load_gather(dlt_v, [di])
            d1 = plsc.load_gather(dlt_v, [di + _TCELL])
            d2 = plsc.load_gather(dlt_v, [di + 2 * _TCELL])
            d3 = plsc.load_gather(dlt_v, [di + 3 * _TCELL])
            px1 = jnp.minimum(jnp.maximum(ax1 + d0, 0.0), _IMG)
            py1 = jnp.minimum(jnp.maximum(ay1 + d1, 0.0), _IMG)
            px2 = jnp.minimum(jnp.maximum(ax2 + d2, 0.0), _IMG)
            py2 = jnp.minimum(jnp.maximum(ay2 + d3, 0.0), _IMG)
            rw = jnp.maximum(ax2 - ax1, 1e-3)
            rh = jnp.maximum(ay2 - ay1, 1e-3)
            rcx = (ax1 + ax2) * 0.5
            rcy = (ay1 + ay2) * 0.5
            pw = jnp.maximum(px2 - px1, 1e-3)
            ph = jnp.maximum(py2 - py1, 1e-3)
            pcx = (px1 + px2) * 0.5
            pcy = (py1 + py2) * 0.5
            gw = jnp.maximum(gx2 - gx1, 1e-3)
            gh = jnp.maximum(gy2 - gy1, 1e-3)
            gcx = (gx1 + gx2) * 0.5
            gcy = (gy1 + gy2) * 0.5
            dx = (pcx - rcx) / rw - (gcx - rcx) / rw
            dy = (pcy - rcy) / rh - (gcy - rcy) / rh
            dw = _vlog(pw / rw) - _vlog(gw / rw)
            dh = _vlog(ph / rh) - _vlog(gh / rh)
            sl1 = jnp.float32(0.0)
            for d in (dx, dy, dw, dh):
                ad = jnp.abs(d)
                sl1 = sl1 + jnp.where(ad < 0.1, 0.5 * d * d / 0.1, ad - 0.05)
            a_bb = a_bb + jnp.where(selp, sl1, 0.0)

            run_p = run_p + jnp.sum(posf)
            run_n = run_n + jnp.sum(negf)
            return run_p, run_n, a_bce, a_bb, a_pr

        z16f = jnp.zeros((_L,), jnp.float32)
        _, _, a_bce, a_bb, a_pr = lax.fori_loop(
            0, _NCHUNK, chunk_b,
            (jnp.float32(0.0), jnp.float32(0.0), z16f, z16f, z16f))

        bce_s = jnp.sum(a_bce)
        bb_s = jnp.sum(a_bb)
        pr_s = jnp.sum(a_pr)
        row_v[...] = jnp.where(lane == 0, bce_s,
                               jnp.where(lane == 1, bb_s,
                                         jnp.where(lane == 2, pr_s, 0.0)))
        pltpu.sync_copy(row_v, prt_sh.at[pl.ds(s * _L, _L)])
        plsc.subcore_barrier()
        pltpu.sync_copy(prt_sh, prt_v)
        tot_bce = jnp.sum(plsc.load_gather(prt_v, [lane16]))
        tot_bb = jnp.sum(plsc.load_gather(prt_v, [lane16 + 1]))
        tot_pr = jnp.sum(plsc.load_gather(prt_v, [lane16 + 2]))

        acc_cls = acc_cls + jnp.full((_L,), tot_bce) / jnp.full((_L,), denom)
        acc_bb = acc_bb + jnp.full((_L,), tot_bb * 0.25)
        acc_bg = acc_bg + jnp.full((_L,), nseln)
        acc_fg = acc_fg + jnp.full((_L,), nselp)
        acc_pr = acc_pr + jnp.full((_L,), tot_pr)
        return acc_cls, acc_bb, acc_bg, acc_fg, acc_pr

    z = jnp.zeros((_L,), jnp.float32)
    acc_cls, acc_bb, acc_bg, acc_fg, acc_pr = lax.fori_loop(
        0, _N // _NC, image_step, (z, z, z, z, z))

    row_v[...] = jnp.where(
        lane == 0, acc_cls,
        jnp.where(lane == 1, acc_bb,
                  jnp.where(lane == 2, acc_bg,
                            jnp.where(lane == 3, acc_fg,
                                      jnp.where(lane == 4, acc_pr, 0.0)))))

    @pl.when(s == 0)
    def _():
        pltpu.sync_copy(row_v, out_h.at[c])


def kernel(cls_scores, bbox_deltas, gt_boxes):
    N, _C, H, W = cls_scores.shape
    stride = int(round(_IMG / float(W)))
    a4 = _gen_anchors(stride, W)

    anc_kc = a4.reshape(_CELLS, _K, 4).transpose(2, 1, 0)
    anc_tc = jnp.asarray(np.ascontiguousarray(anc_kc).reshape(4, _TR, _TCL))

    anc_t = np.zeros((4, _CP * _K), np.float32)
    anc_t[:, :_A] = a4.reshape(_A, 4).T
    anc_p = jnp.asarray(anc_t.reshape(-1))

    cls_k = cls_scores.reshape(N, _K, _CELLS)
    cls_p = jnp.pad(cls_k, ((0, 0), (0, 0), (0, _CP - _CELLS))).reshape(-1)

    dlt_k = bbox_deltas.reshape(N, 4 * _K, _CELLS)
    dlt_p = jnp.pad(dlt_k, ((0, 0), (0, 0), (0, _CP - _CELLS))).reshape(-1)

    dlt_tc = (bbox_deltas.reshape(N, _K, 4, _CELLS)
              .transpose(0, 2, 1, 3).reshape(N, 4, _TR, _TCL))

    gtt = jnp.transpose(gt_boxes, (0, 2, 1)).reshape(-1)
    gt_tc = jnp.pad(gtt.reshape(N, 4 * _M),
                    ((0, 0), (0, 1024 - 4 * _M))).reshape(N, 8, 128)

    mat = _tc_match(anc_tc, dlt_tc, gt_tc)
    mat_p = jnp.pad(mat.reshape(N, _K, _CELLS),
                    ((0, 0), (0, 0), (0, _CP - _CELLS))).reshape(-1)

    o = _sc_kernel(cls_p, dlt_p, anc_p, gtt, mat_p)
    tot_cls = o[0, 0] + o[1, 0]
    tot_bbox = o[0, 1] + o[1, 1]
    tot_bg = o[0, 2] + o[1, 2]
    tot_fg = o[0, 3] + o[1, 3]
    pred_mean = (o[0, 4] + o[1, 4]) / np.float32(188.0)
    return (tot_cls, tot_bbox, tot_bg, tot_fg, pred_mean)


# speedup vs baseline: 1.0882x; 1.0882x over previous
import functools

import numpy as np
import jax
import jax.numpy as jnp
from jax import lax
from jax.experimental import pallas as pl
from jax.experimental.pallas import tpu as pltpu
from jax.experimental.pallas import tpu_sc as plsc

_RATIOS = (0.5, 1.0, 2.0)
_SCALES = (2, 4, 8)
_IMG = 1920.0
_L = 16
_NC = 2
_NS = 16
_N = 4
_M = 64
_K = 9
_G = 60
_CELLS = _G * _G
_CP = 4096
_A = _CELLS * _K
_TCELL = _CP // _NS
_CH = _TCELL * _K
_NCHUNK = _CH // _L
_TR = 72
_TCL = 450


def _gen_anchors(stride, G):
    ws, hs = [], []
    for r in _RATIOS:
        for s in _SCALES:
            size = stride * s
            ws.append(size * np.sqrt(1.0 / r))
            hs.append(size * np.sqrt(r))
    ws = np.asarray(ws, np.float32)
    hs = np.asarray(hs, np.float32)
    ctr = (np.arange(G, dtype=np.float32) + 0.5) * stride
    cy, cx = np.meshgrid(ctr, ctr, indexing='ij')
    cx = cx[:, :, None]
    cy = cy[:, :, None]
    x1 = cx - ws / 2.0
    y1 = cy - hs / 2.0
    x2 = cx + ws / 2.0
    y2 = cy + hs / 2.0
    return np.stack([x1, y1, x2, y2], axis=-1).astype(np.float32)


def _vlog(x):
    b = lax.bitcast_convert_type(x, jnp.int32)
    e = (b >> 23) - 127
    m = lax.bitcast_convert_type((b & 0x7FFFFF) | 0x3F800000, jnp.float32)
    big = m > 1.4142135623730951
    m = jnp.where(big, m * 0.5, m)
    ef = jnp.where(big, e + 1, e).astype(jnp.float32)
    s = (m - 1.0) / (m + 1.0)
    z = s * s
    p = 1.0 / 3 + z * (1.0 / 5 + z * (1.0 / 7 + z * (1.0 / 9)))
    return (2.0 * s) * (1.0 + z * p) + ef * 0.6931471805599453


def _tc_match_kernel(anc_ref, dlt_ref, gt_ref, mat_ref):
    rx1 = jnp.clip(anc_ref[0] + dlt_ref[0, 0], 0.0, _IMG)
    ry1 = jnp.clip(anc_ref[1] + dlt_ref[0, 1], 0.0, _IMG)
    rx2 = jnp.clip(anc_ref[2] + dlt_ref[0, 2], 0.0, _IMG)
    ry2 = jnp.clip(anc_ref[3] + dlt_ref[0, 3], 0.0, _IMG)
    area_r = (rx2 - rx1) * (ry2 - ry1)
    best = jnp.full((_TR, _TCL), -1.0, jnp.float32)
    bidx = jnp.zeros((_TR, _TCL), jnp.int32)
    for m in range(_M):
        gx1 = gt_ref[0, m // 128, m % 128]
        gy1 = gt_ref[0, (_M + m) // 128, (_M + m) % 128]
        gx2 = gt_ref[0, (2 * _M + m) // 128, (2 * _M + m) % 128]
        gy2 = gt_ref[0, (3 * _M + m) // 128, (3 * _M + m) % 128]
        garea = (gx2 - gx1) * (gy2 - gy1)
        xx1 = jnp.maximum(rx1, gx1)
        yy1 = jnp.maximum(ry1, gy1)
        xx2 = jnp.minimum(rx2, gx2)
        yy2 = jnp.minimum(ry2, gy2)
        w = jnp.maximum(xx2 - xx1, 0.0)
        h = jnp.maximum(yy2 - yy1, 0.0)
        inter = w * h
        iou = inter / (area_r + garea - inter + 1e-9)
        upd = iou > best
        best = jnp.where(upd, iou, best)
        bidx = jnp.where(upd, jnp.int32(m), bidx)
    mat_ref[0] = jnp.where(best >= 0.4, bidx,
                           jnp.where(best < 0.1, jnp.int32(-1), jnp.int32(-2)))


def _make_tc(i0):
    return pl.pallas_call(
        _tc_match_kernel,
        grid=(2,),
        in_specs=[
            pl.BlockSpec((4, _TR, _TCL), lambda n: (0, 0, 0)),
            pl.BlockSpec((1, 4, _TR, _TCL), lambda n: (n + i0, 0, 0, 0)),
            pl.BlockSpec((1, 8, 128), lambda n: (n + i0, 0, 0)),
        ],
        out_specs=pl.BlockSpec((1, _TR, _TCL), lambda n: (n, 0, 0)),
        out_shape=jax.ShapeDtypeStruct((2, _TR, _TCL), jnp.int32),
    )


_tc_match01 = _make_tc(0)
_tc_match23 = _make_tc(2)


_mesh = plsc.VectorSubcoreMesh(core_axis_name="c", subcore_axis_name="s",
                               num_cores=_NC)


def _make_sc(i0):
  @functools.partial(
    pl.kernel,
    mesh=_mesh,
    compiler_params=pltpu.CompilerParams(needs_layout_passes=False),
    out_type=jax.ShapeDtypeStruct((_NC, _L), jnp.float32),
    scratch_types=[
        pltpu.VMEM((_K * _TCELL,), jnp.float32),
        pltpu.VMEM((4 * _K * _TCELL,), jnp.float32),
        pltpu.VMEM((4, _CH), jnp.float32),
        pltpu.VMEM((4 * _M,), jnp.float32),
        pltpu.VMEM((_K * _TCELL,), jnp.int32),
        pltpu.VMEM((_L,), jnp.float32),
        pltpu.VMEM((_NS * _L,), jnp.float32),
        pltpu.VMEM((_NS * _L,), jnp.float32),
        pltpu.VMEM_SHARED((_NS * _L,), jnp.float32),
        pltpu.VMEM_SHARED((_NS * _L,), jnp.float32),
    ],
  )
  def _sc_kernel(cls_h, dlt_h, anc_h, gtt_h, mat_h, out_h,
                 cls_v, dlt_v, anc_v, gt_v, mat_v, row_v,
                 cnt_v, prt_v, cnt_sh, prt_sh):
    c = lax.axis_index("c")
    s = lax.axis_index("s")
    base_a = s * _CH
    c0 = s * _TCELL
    lane = lax.iota(jnp.int32, _L)

    for j in range(4):
        pltpu.sync_copy(anc_h.at[pl.ds(j * (_CP * _K) + base_a, _CH)],
                        anc_v.at[j])

    def image_step(t_img, accs):
        acc_cls, acc_bb, acc_bg, acc_fg, acc_pr = accs
        img = i0 + c
        is_last = img == _N - 1

        for k in range(_K):
            pltpu.sync_copy(
                cls_h.at[pl.ds((img * _K + k) * _CP + c0, _TCELL)],
                cls_v.at[pl.ds(k * _TCELL, _TCELL)])
            pltpu.sync_copy(
                mat_h.at[pl.ds((c * _K + k) * _CP + c0, _TCELL)],
                mat_v.at[pl.ds(k * _TCELL, _TCELL)])
        for r in range(4 * _K):
            pltpu.sync_copy(
                dlt_h.at[pl.ds((img * 4 * _K + r) * _CP + c0, _TCELL)],
                dlt_v.at[pl.ds(r * _TCELL, _TCELL)])
        for j in range(4):
            pltpu.sync_copy(gtt_h.at[pl.ds((img * 4 + j) * _M, _M)],
                            gt_v.at[pl.ds(j * _M, _M)])

        def perm_idx(o):
            ti = o + lane
            tf = ti.astype(jnp.float32)
            cellr = ((tf + 0.5) * (1.0 / 9.0)).astype(jnp.int32)
            ki = ti - cellr * 9
            ci = ki * _TCELL + cellr
            di = ki * (4 * _TCELL) + cellr
            return ci, di

        def chunk_a(t, carry):
            cp, cn = carry
            o = t * _L
            ci, _ = perm_idx(o)
            match = plsc.load_gather(mat_v, [ci])
            valid = (base_a + o + lane) < _A
            cp = cp + jnp.sum(jnp.where((match >= 0) & valid, 1.0, 0.0))
            cn = cn + jnp.sum(jnp.where((match == -1) & valid, 1.0, 0.0))
            return cp, cn

        cp, cn = lax.fori_loop(0, _NCHUNK, chunk_a,
                               (jnp.float32(0.0), jnp.float32(0.0)))

        row_v[...] = jnp.where(lane == 0, cp, jnp.where(lane == 1, cn, 0.0))
        pltpu.sync_copy(row_v, cnt_sh.at[pl.ds(s * _L, _L)])
        plsc.subcore_barrier()
        pltpu.sync_copy(cnt_sh, cnt_v)

        lane16 = lane * _L
        cpos_all = plsc.load_gather(cnt_v, [lane16])
        cneg_all = plsc.load_gather(cnt_v, [lane16 + 1])
        before = lane < s
        base_pos = jnp.sum(jnp.where(before, cpos_all, 0.0))
        base_neg = jnp.sum(jnp.where(before, cneg_all, 0.0))
        totpos = jnp.sum(cpos_all)
        totneg = jnp.sum(cneg_all)
        nselp = jnp.minimum(totpos, 128.0)
        nseln = jnp.minimum(totneg, 60.0)
        denom = jnp.maximum(nselp + nseln, 1.0)
        capp = 128.0 - nselp
        capn = 60.0 - nseln

        def chunk_b(t, carry):
            run_p, run_n, a_bce, a_bb, a_pr = carry
            o = t * _L
            ci, di = perm_idx(o)
            match = plsc.load_gather(mat_v, [ci])
            valid = (base_a + o + lane) < _A
            pos = (match >= 0) & valid
            neg = (match == -1) & valid
            posf = jnp.where(pos, 1.0, 0.0)
            negf = jnp.where(neg, 1.0, 0.0)
            rp = run_p + (plsc.cumsum(posf) - posf)
            rn = run_n + (plsc.cumsum(negf) - negf)
            g = (base_a + o + lane).astype(jnp.float32)
            gr_pos = base_pos + rp
            gr_neg = base_neg + rn
            gr_npos = g - gr_pos
            gr_nneg = g - gr_neg
            selp = pos & (gr_pos < 128.0)
            seln = neg & (gr_neg < 60.0)
            padp = jnp.logical_and((~pos) & (gr_npos < capp), is_last)
            padn = jnp.logical_and((~neg) & (gr_nneg < capn), is_last)
            selp_l = jnp.logical_and(selp, is_last)
            seln_l = jnp.logical_and(seln, is_last)

            p = plsc.load_gather(cls_v, [ci])
            u = _vlog(1.0 + jnp.exp(-jnp.abs(p)))
            a_bce = a_bce + jnp.where(selp, jnp.maximum(-p, 0.0) + u, 0.0) \
                          + jnp.where(seln, jnp.maximum(p, 0.0) + u, 0.0)
            a_pr = a_pr + jnp.where(selp_l, p, 0.0) + jnp.where(padp, p, 0.0) \
                        + jnp.where(seln_l, p, 0.0) + jnp.where(padn, p, 0.0)

            midx = jnp.maximum(match, 0)
            gx1 = plsc.load_gather(gt_v, [midx])
            gy1 = plsc.load_gather(gt_v, [midx + _M])
            gx2 = plsc.load_gather(gt_v, [midx + 2 * _M])
            gy2 = plsc.load_gather(gt_v, [midx + 3 * _M])
            ax1 = anc_v[0, pl.ds(o, _L)]
            ay1 = anc_v[1, pl.ds(o, _L)]
            ax2 = anc_v[2, pl.ds(o, _L)]
            ay2 = anc_v[3, pl.ds(o, _L)]
            d0 = plsc.load_gather(dlt_v, [di])
            d1 = plsc.load_gather(dlt_v, [di + _TCELL])
            d2 = plsc.load_gather(dlt_v, [di + 2 * _TCELL])
            d3 = plsc.load_gather(dlt_v, [di + 3 * _TCELL])
            px1 = jnp.minimum(jnp.maximum(ax1 + d0, 0.0), _IMG)
            py1 = jnp.minimum(jnp.maximum(ay1 + d1, 0.0), _IMG)
            px2 = jnp.minimum(jnp.maximum(ax2 + d2, 0.0), _IMG)
            py2 = jnp.minimum(jnp.maximum(ay2 + d3, 0.0), _IMG)
            rw = jnp.maximum(ax2 - ax1, 1e-3)
            rh = jnp.maximum(ay2 - ay1, 1e-3)
            rcx = (ax1 + ax2) * 0.5
            rcy = (ay1 + ay2) * 0.5
            pw = jnp.maximum(px2 - px1, 1e-3)
            ph = jnp.maximum(py2 - py1, 1e-3)
            pcx = (px1 + px2) * 0.5
            pcy = (py1 + py2) * 0.5
            gw = jnp.maximum(gx2 - gx1, 1e-3)
            gh = jnp.maximum(gy2 - gy1, 1e-3)
            gcx = (gx1 + gx2) * 0.5
            gcy = (gy1 + gy2) * 0.5
            dx = (pcx - rcx) / rw - (gcx - rcx) / rw
            dy = (pcy - rcy) / rh - (gcy - rcy) / rh
            dw = _vlog(pw / rw) - _vlog(gw / rw)
            dh = _vlog(ph / rh) - _vlog(gh / rh)
            sl1 = jnp.float32(0.0)
            for d in (dx, dy, dw, dh):
                ad = jnp.abs(d)
                sl1 = sl1 + jnp.where(ad < 0.1, 0.5 * d * d / 0.1, ad - 0.05)
            a_bb = a_bb + jnp.where(selp, sl1, 0.0)

            run_p = run_p + jnp.sum(posf)
            run_n = run_n + jnp.sum(negf)
            return run_p, run_n, a_bce, a_bb, a_pr

        z16f = jnp.zeros((_L,), jnp.float32)
        _, _, a_bce, a_bb, a_pr = lax.fori_loop(
            0, _NCHUNK, chunk_b,
            (jnp.float32(0.0), jnp.float32(0.0), z16f, z16f, z16f))

        bce_s = jnp.sum(a_bce)
        bb_s = jnp.sum(a_bb)
        pr_s = jnp.sum(a_pr)
        row_v[...] = jnp.where(lane == 0, bce_s,
                               jnp.where(lane == 1, bb_s,
                                         jnp.where(lane == 2, pr_s, 0.0)))
        pltpu.sync_copy(row_v, prt_sh.at[pl.ds(s * _L, _L)])
        plsc.subcore_barrier()
        pltpu.sync_copy(prt_sh, prt_v)
        tot_bce = jnp.sum(plsc.load_gather(prt_v, [lane16]))
        tot_bb = jnp.sum(plsc.load_gather(prt_v, [lane16 + 1]))
        tot_pr = jnp.sum(plsc.load_gather(prt_v, [lane16 + 2]))

        acc_cls = acc_cls + jnp.full((_L,), tot_bce) / jnp.full((_L,), denom)
        acc_bb = acc_bb + jnp.full((_L,), tot_bb * 0.25)
        acc_bg = acc_bg + jnp.full((_L,), nseln)
        acc_fg = acc_fg + jnp.full((_L,), nselp)
        acc_pr = acc_pr + jnp.full((_L,), tot_pr)
        return acc_cls, acc_bb, acc_bg, acc_fg, acc_pr

    z = jnp.zeros((_L,), jnp.float32)
    acc_cls, acc_bb, acc_bg, acc_fg, acc_pr = lax.fori_loop(
        0, 1, image_step, (z, z, z, z, z))

    row_v[...] = jnp.where(
        lane == 0, acc_cls,
        jnp.where(lane == 1, acc_bb,
                  jnp.where(lane == 2, acc_bg,
                            jnp.where(lane == 3, acc_fg,
                                      jnp.where(lane == 4, acc_pr, 0.0)))))

    @pl.when(s == 0)
    def _():
        pltpu.sync_copy(row_v, out_h.at[c])

  return _sc_kernel


_sc_kernel01 = _make_sc(0)
_sc_kernel23 = _make_sc(2)


def kernel(cls_scores, bbox_deltas, gt_boxes):
    N, _C, H, W = cls_scores.shape
    stride = int(round(_IMG / float(W)))
    a4 = _gen_anchors(stride, W)

    anc_kc = a4.reshape(_CELLS, _K, 4).transpose(2, 1, 0)
    anc_tc = jnp.asarray(np.ascontiguousarray(anc_kc).reshape(4, _TR, _TCL))

    anc_t = np.zeros((4, _CP * _K), np.float32)
    anc_t[:, :_A] = a4.reshape(_A, 4).T
    anc_p = jnp.asarray(anc_t.reshape(-1))

    cls_k = cls_scores.reshape(N, _K, _CELLS)
    cls_p = jnp.pad(cls_k, ((0, 0), (0, 0), (0, _CP - _CELLS))).reshape(-1)

    dlt_k = bbox_deltas.reshape(N, 4 * _K, _CELLS)
    dlt_p = jnp.pad(dlt_k, ((0, 0), (0, 0), (0, _CP - _CELLS))).reshape(-1)

    dlt_tc = (bbox_deltas.reshape(N, _K, 4, _CELLS)
              .transpose(0, 2, 1, 3).reshape(N, 4, _TR, _TCL))

    gtt = jnp.transpose(gt_boxes, (0, 2, 1)).reshape(-1)
    gt_tc = jnp.pad(gtt.reshape(N, 4 * _M),
                    ((0, 0), (0, 1024 - 4 * _M))).reshape(N, 8, 128)

    mat01 = _tc_match01(anc_tc, dlt_tc, gt_tc)
    mat23 = _tc_match23(anc_tc, dlt_tc, gt_tc)
    mat01_p = jnp.pad(mat01.reshape(2, _K, _CELLS),
                      ((0, 0), (0, 0), (0, _CP - _CELLS))).reshape(-1)
    mat23_p = jnp.pad(mat23.reshape(2, _K, _CELLS),
                      ((0, 0), (0, 0), (0, _CP - _CELLS))).reshape(-1)

    o1 = _sc_kernel01(cls_p, dlt_p, anc_p, gtt, mat01_p)
    o2 = _sc_kernel23(cls_p, dlt_p, anc_p, gtt, mat23_p)
    o = o1 + o2
    tot_cls = o[0, 0] + o[1, 0]
    tot_bbox = o[0, 1] + o[1, 1]
    tot_bg = o[0, 2] + o[1, 2]
    tot_fg = o[0, 3] + o[1, 3]
    pred_mean = (o[0, 4] + o[1, 4]) / np.float32(188.0)
    return (tot_cls, tot_bbox, tot_bg, tot_fg, pred_mean)
